# Initial kernel scaffold; baseline (speedup 1.0000x reference)
import jax
import jax.numpy as jnp
from jax.experimental import pallas as pl


def kernel(x, edge_index, edge_attr, W1, b1, gamma, beta, W2, b2):
    raise NotImplementedError("write your pallas kernel here")


# speedup vs baseline: 8.8437x; 8.8437x over previous
import functools

import jax
import jax.numpy as jnp
from jax import lax
from jax.experimental import pallas as pl
from jax.experimental.pallas import tpu as pltpu
from jax.experimental.pallas import tpu_sc as plsc

N = 10000
NP = 10240
E = 320000
NW = 32
CH = 79
K = 128
EP = NW * CH * K
RT = NP // 16
D_FEAT = 128
BLK = 2560

_MESH = plsc.VectorSubcoreMesh(
    core_axis_name="c", subcore_axis_name="s", num_cores=2, num_subcores=16)


def _zero_rows(rows):
    def body(i, _):
        for j in range(8):
            rows[i, pl.ds(j * 16, 16)] = jnp.zeros((16,), jnp.float32)
        return 0
    lax.fori_loop(0, 128, body, 0)


def _rowpass_body(src, gidx, sidx, out, gv, sv, rows, acc):
    c = lax.axis_index("c")
    s = lax.axis_index("s")
    wid = s * 2 + c
    pltpu.sync_copy(gidx.at[wid], gv)
    pltpu.sync_copy(sidx.at[wid], sv)
    _zero_rows(rows)
    base = s * RT
    for k in range(RT // K):
        pltpu.sync_copy(rows, acc.at[pl.ds(base + k * K, K)])
    plsc.subcore_barrier()

    def chunk(ci, _):
        pltpu.sync_copy(src.at[gv.at[ci]], rows)
        pltpu.sync_copy(rows, acc.at[sv.at[ci]], add=True)
        return 0
    lax.fori_loop(0, CH, chunk, 0)
    plsc.subcore_barrier()
    pltpu.sync_copy(acc.at[pl.ds(base, RT)], out.at[c, pl.ds(base, RT)])


_sc_rowpass = pl.kernel(
    _rowpass_body,
    out_type=jax.ShapeDtypeStruct((2, NP, D_FEAT), jnp.float32),
    mesh=_MESH,
    scratch_types=[
        pltpu.VMEM((CH, K), jnp.int32),
        pltpu.VMEM((CH, K), jnp.int32),
        pltpu.VMEM((K, D_FEAT), jnp.float32),
        pltpu.VMEM_SHARED((NP, D_FEAT), jnp.float32),
    ],
)


def _degrees_body(nidx, hidx, attr, out, nv, hv, vals, ones, zb, accd, accb):
    c = lax.axis_index("c")
    s = lax.axis_index("s")
    wid = s * 2 + c
    pltpu.sync_copy(nidx.at[wid], nv)
    pltpu.sync_copy(hidx.at[wid], hv)

    def zset(i, _):
        zb[pl.ds(i * 16, 16)] = jnp.zeros((16,), jnp.float32)
        return 0
    lax.fori_loop(0, 40, zset, 0)

    def oset(i, _):
        ones[pl.ds(i * 16, 16)] = jnp.full((16,), 1.0, jnp.float32)
        return 0
    lax.fori_loop(0, 8, oset, 0)

    base = s * RT
    pltpu.sync_copy(zb.at[pl.ds(0, RT)], accd.at[pl.ds(base, RT)])
    pltpu.sync_copy(zb.at[pl.ds(0, RT)], accb.at[pl.ds(base, RT)])
    plsc.subcore_barrier()

    def chunk(ci, _):
        pltpu.sync_copy(attr.at[hv.at[ci]], vals)
        pltpu.sync_copy(vals, accd.at[nv.at[ci]], add=True)
        pltpu.sync_copy(ones, accb.at[hv.at[ci]], add=True)
        return 0
    lax.fori_loop(0, CH, chunk, 0)
    plsc.subcore_barrier()
    pltpu.sync_copy(accd.at[pl.ds(base, RT)], out.at[c, 0, pl.ds(base, RT)])
    pltpu.sync_copy(accb.at[pl.ds(base, RT)], out.at[c, 1, pl.ds(base, RT)])


_sc_degrees = pl.kernel(
    _degrees_body,
    out_type=jax.ShapeDtypeStruct((2, 2, NP), jnp.float32),
    mesh=_MESH,
    scratch_types=[
        pltpu.VMEM((CH, K), jnp.int32),
        pltpu.VMEM((CH, K), jnp.int32),
        pltpu.VMEM((K,), jnp.float32),
        pltpu.VMEM((K,), jnp.float32),
        pltpu.VMEM((640,), jnp.float32),
        pltpu.VMEM_SHARED((NP,), jnp.float32),
        pltpu.VMEM_SHARED((NP,), jnp.float32),
    ],
)


def _safe_inv(d):
    return jnp.where(d == 0, 0.0, 1.0 / jnp.where(d == 0, 1.0, d))


def _mm_body(x_ref, w_ref, o_ref):
    o_ref[...] = jnp.dot(x_ref[...], w_ref[...],
                         preferred_element_type=jnp.float32)


_tc_matmul = pl.pallas_call(
    _mm_body,
    grid=(NP // BLK,),
    in_specs=[
        pl.BlockSpec((BLK, D_FEAT), lambda i: (i, 0)),
        pl.BlockSpec((D_FEAT, D_FEAT), lambda i: (0, 0)),
    ],
    out_specs=pl.BlockSpec((BLK, D_FEAT), lambda i: (i, 0)),
    out_shape=jax.ShapeDtypeStruct((NP, D_FEAT), jnp.float32),
)


def _scale_body(e_ref, db_ref, o_ref):
    ssum = e_ref[0] + e_ref[1]
    b = db_ref[0, 1] + db_ref[1, 1]
    o_ref[...] = ssum * _safe_inv(b)


_tc_scale = pl.pallas_call(
    _scale_body,
    grid=(NP // BLK,),
    in_specs=[
        pl.BlockSpec((2, BLK, D_FEAT), lambda i: (0, i, 0)),
        pl.BlockSpec((2, 2, BLK, 1), lambda i: (0, 0, i, 0)),
    ],
    out_specs=pl.BlockSpec((BLK, D_FEAT), lambda i: (i, 0)),
    out_shape=jax.ShapeDtypeStruct((NP, D_FEAT), jnp.float32),
)


def _fuse_body(p_ref, db_ref, b1_ref, g_ref, bt_ref, w_ref, o_ref):
    ssum = p_ref[0] + p_ref[1]
    d = db_ref[0, 0] + db_ref[1, 0]
    h = ssum * _safe_inv(d) + b1_ref[...]
    h = jnp.maximum(h, 0.0)
    mu = jnp.mean(h, axis=-1, keepdims=True)
    var = jnp.mean((h - mu) ** 2, axis=-1, keepdims=True)
    hn = (h - mu) / jnp.sqrt(var + 1e-5) * g_ref[...] + bt_ref[...]
    o_ref[...] = jnp.dot(hn, w_ref[...], preferred_element_type=jnp.float32)


_tc_fuse = pl.pallas_call(
    _fuse_body,
    grid=(NP // BLK,),
    in_specs=[
        pl.BlockSpec((2, BLK, D_FEAT), lambda i: (0, i, 0)),
        pl.BlockSpec((2, 2, BLK, 1), lambda i: (0, 0, i, 0)),
        pl.BlockSpec((1, D_FEAT), lambda i: (0, 0)),
        pl.BlockSpec((1, D_FEAT), lambda i: (0, 0)),
        pl.BlockSpec((1, D_FEAT), lambda i: (0, 0)),
        pl.BlockSpec((D_FEAT, D_FEAT), lambda i: (0, 0)),
    ],
    out_specs=pl.BlockSpec((BLK, D_FEAT), lambda i: (i, 0)),
    out_shape=jax.ShapeDtypeStruct((NP, D_FEAT), jnp.float32),
)


def _final_body(p_ref, db_ref, b2_ref, x_ref, o_ref):
    ssum = p_ref[0] + p_ref[1]
    d = db_ref[0, 0] + db_ref[1, 0]
    o_ref[...] = ssum * _safe_inv(d) + b2_ref[...] + x_ref[...]


_tc_final = pl.pallas_call(
    _final_body,
    grid=(NP // BLK,),
    in_specs=[
        pl.BlockSpec((2, BLK, D_FEAT), lambda i: (0, i, 0)),
        pl.BlockSpec((2, 2, BLK, 1), lambda i: (0, 0, i, 0)),
        pl.BlockSpec((1, D_FEAT), lambda i: (0, 0)),
        pl.BlockSpec((BLK, D_FEAT), lambda i: (i, 0)),
    ],
    out_specs=pl.BlockSpec((BLK, D_FEAT), lambda i: (i, 0)),
    out_shape=jax.ShapeDtypeStruct((NP, D_FEAT), jnp.float32),
)


def kernel(x, edge_index, edge_attr, W1, b1, gamma, beta, W2, b2):
    xp = jnp.pad(x, ((0, NP - N), (0, 0)))
    attr_p = jnp.pad(edge_attr, (0, NP - N))
    nidx = jnp.pad(edge_index[0], (0, EP - E), constant_values=N).reshape(NW, CH, K)
    hidx = jnp.pad(edge_index[1], (0, EP - E), constant_values=N).reshape(NW, CH, K)
    b1r = b1.reshape(1, D_FEAT)
    b2r = b2.reshape(1, D_FEAT)
    gr = gamma.reshape(1, D_FEAT)
    btr = beta.reshape(1, D_FEAT)

    db = _sc_degrees(nidx, hidx, attr_p).reshape(2, 2, NP, 1)
    xl1 = _tc_matmul(xp, W1)
    e1 = _tc_scale(_sc_rowpass(xl1, nidx, hidx), db)
    xl2 = _tc_fuse(_sc_rowpass(e1, hidx, nidx), db, b1r, gr, btr, W2)
    e2 = _tc_scale(_sc_rowpass(xl2, nidx, hidx), db)
    out = _tc_final(_sc_rowpass(e2, hidx, nidx), db, b2r, xp)
    return out[:N]


# speedup vs baseline: 10.5337x; 1.1911x over previous
import functools

import jax
import jax.numpy as jnp
from jax import lax
from jax.experimental import pallas as pl
from jax.experimental.pallas import tpu as pltpu
from jax.experimental.pallas import tpu_sc as plsc

N = 10000
NP = 10240
E = 320000
NW = 32
CH = 79
K = 128
EP = NW * CH * K
RT = NP // 16
D_FEAT = 128
BLK = 2560

_MESH = plsc.VectorSubcoreMesh(
    core_axis_name="c", subcore_axis_name="s", num_cores=2, num_subcores=16)


def _zero_rows(rows):
    def body(i, _):
        for j in range(8):
            rows[i, pl.ds(j * 16, 16)] = jnp.zeros((16,), jnp.float32)
        return 0
    lax.fori_loop(0, 128, body, 0)


def _rowpass_body(src, gidx, sidx, out, gvb, svb, rows, acc, gsem, isem):
    c = lax.axis_index("c")
    s = lax.axis_index("s")
    wid = s * 2 + c
    gw = gidx.at[wid]
    sw = sidx.at[wid]
    _zero_rows(rows.at[0])
    base = s * RT
    for k in range(RT // K):
        pltpu.sync_copy(rows.at[0], acc.at[pl.ds(base + k * K, K)])
    plsc.subcore_barrier()

    pltpu.sync_copy(gw.at[0], gvb.at[0])
    pltpu.sync_copy(sw.at[0], svb.at[0])
    pltpu.async_copy(src.at[gvb.at[0]], rows.at[0], gsem.at[0])
    if CH > 1:
        pltpu.async_copy(gw.at[1], gvb.at[1], isem.at[1])
        pltpu.async_copy(sw.at[1], svb.at[1], isem.at[1])

    def chunk(ci, _):
        p = lax.rem(ci, 2)
        pn = 1 - p

        @pl.when(ci + 1 < CH)
        def _():
            pltpu.make_async_copy(gw.at[ci + 1], gvb.at[pn], isem.at[pn]).wait()
            pltpu.make_async_copy(sw.at[ci + 1], svb.at[pn], isem.at[pn]).wait()
            pltpu.async_copy(src.at[gvb.at[pn]], rows.at[pn], gsem.at[pn])

        pltpu.make_async_copy(src.at[gvb.at[p]], rows.at[p], gsem.at[p]).wait()
        pltpu.sync_copy(rows.at[p], acc.at[svb.at[p]], add=True)

        @pl.when(ci + 2 < CH)
        def _():
            pltpu.async_copy(gw.at[ci + 2], gvb.at[p], isem.at[p])
            pltpu.async_copy(sw.at[ci + 2], svb.at[p], isem.at[p])
        return 0
    lax.fori_loop(0, CH, chunk, 0)
    plsc.subcore_barrier()
    pltpu.sync_copy(acc.at[pl.ds(base, RT)], out.at[c, pl.ds(base, RT)])


_sc_rowpass = pl.kernel(
    _rowpass_body,
    out_type=jax.ShapeDtypeStruct((2, NP, D_FEAT), jnp.float32),
    mesh=_MESH,
    scratch_types=[
        pltpu.VMEM((2, K), jnp.int32),
        pltpu.VMEM((2, K), jnp.int32),
        pltpu.VMEM((2, K, D_FEAT), jnp.float32),
        pltpu.VMEM_SHARED((NP, D_FEAT), jnp.float32),
        pltpu.SemaphoreType.DMA((2,)),
        pltpu.SemaphoreType.DMA((2,)),
    ],
)


def _degrees_body(nidx, hidx, attr, out, nv, hv, vals, ones, zb, accd, accb):
    c = lax.axis_index("c")
    s = lax.axis_index("s")
    wid = s * 2 + c
    pltpu.sync_copy(nidx.at[wid], nv)
    pltpu.sync_copy(hidx.at[wid], hv)

    def zset(i, _):
        zb[pl.ds(i * 16, 16)] = jnp.zeros((16,), jnp.float32)
        return 0
    lax.fori_loop(0, 40, zset, 0)

    def oset(i, _):
        ones[pl.ds(i * 16, 16)] = jnp.full((16,), 1.0, jnp.float32)
        return 0
    lax.fori_loop(0, 8, oset, 0)

    base = s * RT
    pltpu.sync_copy(zb.at[pl.ds(0, RT)], accd.at[pl.ds(base, RT)])
    pltpu.sync_copy(zb.at[pl.ds(0, RT)], accb.at[pl.ds(base, RT)])
    plsc.subcore_barrier()

    def chunk(ci, _):
        pltpu.sync_copy(attr.at[hv.at[ci]], vals)
        pltpu.sync_copy(vals, accd.at[nv.at[ci]], add=True)
        pltpu.sync_copy(ones, accb.at[hv.at[ci]], add=True)
        return 0
    lax.fori_loop(0, CH, chunk, 0)
    plsc.subcore_barrier()
    pltpu.sync_copy(accd.at[pl.ds(base, RT)], out.at[c, 0, pl.ds(base, RT)])
    pltpu.sync_copy(accb.at[pl.ds(base, RT)], out.at[c, 1, pl.ds(base, RT)])


_sc_degrees = pl.kernel(
    _degrees_body,
    out_type=jax.ShapeDtypeStruct((2, 2, NP), jnp.float32),
    mesh=_MESH,
    scratch_types=[
        pltpu.VMEM((CH, K), jnp.int32),
        pltpu.VMEM((CH, K), jnp.int32),
        pltpu.VMEM((K,), jnp.float32),
        pltpu.VMEM((K,), jnp.float32),
        pltpu.VMEM((640,), jnp.float32),
        pltpu.VMEM_SHARED((NP,), jnp.float32),
        pltpu.VMEM_SHARED((NP,), jnp.float32),
    ],
)


def _safe_inv(d):
    return jnp.where(d == 0, 0.0, 1.0 / jnp.where(d == 0, 1.0, d))


def _mm_body(x_ref, w_ref, o_ref):
    o_ref[...] = jnp.dot(x_ref[...], w_ref[...],
                         preferred_element_type=jnp.float32)


_tc_matmul = pl.pallas_call(
    _mm_body,
    grid=(NP // BLK,),
    in_specs=[
        pl.BlockSpec((BLK, D_FEAT), lambda i: (i, 0)),
        pl.BlockSpec((D_FEAT, D_FEAT), lambda i: (0, 0)),
    ],
    out_specs=pl.BlockSpec((BLK, D_FEAT), lambda i: (i, 0)),
    out_shape=jax.ShapeDtypeStruct((NP, D_FEAT), jnp.float32),
)


def _scale_body(e_ref, db_ref, o_ref):
    ssum = e_ref[0] + e_ref[1]
    b = db_ref[0, 1] + db_ref[1, 1]
    o_ref[...] = ssum * _safe_inv(b)


_tc_scale = pl.pallas_call(
    _scale_body,
    grid=(NP // BLK,),
    in_specs=[
        pl.BlockSpec((2, BLK, D_FEAT), lambda i: (0, i, 0)),
        pl.BlockSpec((2, 2, BLK, 1), lambda i: (0, 0, i, 0)),
    ],
    out_specs=pl.BlockSpec((BLK, D_FEAT), lambda i: (i, 0)),
    out_shape=jax.ShapeDtypeStruct((NP, D_FEAT), jnp.float32),
)


def _fuse_body(p_ref, db_ref, b1_ref, g_ref, bt_ref, w_ref, o_ref):
    ssum = p_ref[0] + p_ref[1]
    d = db_ref[0, 0] + db_ref[1, 0]
    h = ssum * _safe_inv(d) + b1_ref[...]
    h = jnp.maximum(h, 0.0)
    mu = jnp.mean(h, axis=-1, keepdims=True)
    var = jnp.mean((h - mu) ** 2, axis=-1, keepdims=True)
    hn = (h - mu) / jnp.sqrt(var + 1e-5) * g_ref[...] + bt_ref[...]
    o_ref[...] = jnp.dot(hn, w_ref[...], preferred_element_type=jnp.float32)


_tc_fuse = pl.pallas_call(
    _fuse_body,
    grid=(NP // BLK,),
    in_specs=[
        pl.BlockSpec((2, BLK, D_FEAT), lambda i: (0, i, 0)),
        pl.BlockSpec((2, 2, BLK, 1), lambda i: (0, 0, i, 0)),
        pl.BlockSpec((1, D_FEAT), lambda i: (0, 0)),
        pl.BlockSpec((1, D_FEAT), lambda i: (0, 0)),
        pl.BlockSpec((1, D_FEAT), lambda i: (0, 0)),
        pl.BlockSpec((D_FEAT, D_FEAT), lambda i: (0, 0)),
    ],
    out_specs=pl.BlockSpec((BLK, D_FEAT), lambda i: (i, 0)),
    out_shape=jax.ShapeDtypeStruct((NP, D_FEAT), jnp.float32),
)


def _final_body(p_ref, db_ref, b2_ref, x_ref, o_ref):
    ssum = p_ref[0] + p_ref[1]
    d = db_ref[0, 0] + db_ref[1, 0]
    o_ref[...] = ssum * _safe_inv(d) + b2_ref[...] + x_ref[...]


_tc_final = pl.pallas_call(
    _final_body,
    grid=(NP // BLK,),
    in_specs=[
        pl.BlockSpec((2, BLK, D_FEAT), lambda i: (0, i, 0)),
        pl.BlockSpec((2, 2, BLK, 1), lambda i: (0, 0, i, 0)),
        pl.BlockSpec((1, D_FEAT), lambda i: (0, 0)),
        pl.BlockSpec((BLK, D_FEAT), lambda i: (i, 0)),
    ],
    out_specs=pl.BlockSpec((BLK, D_FEAT), lambda i: (i, 0)),
    out_shape=jax.ShapeDtypeStruct((NP, D_FEAT), jnp.float32),
)


def kernel(x, edge_index, edge_attr, W1, b1, gamma, beta, W2, b2):
    xp = jnp.pad(x, ((0, NP - N), (0, 0)))
    attr_p = jnp.pad(edge_attr, (0, NP - N))
    nidx = jnp.pad(edge_index[0], (0, EP - E), constant_values=N).reshape(NW, CH, K)
    hidx = jnp.pad(edge_index[1], (0, EP - E), constant_values=N).reshape(NW, CH, K)
    b1r = b1.reshape(1, D_FEAT)
    b2r = b2.reshape(1, D_FEAT)
    gr = gamma.reshape(1, D_FEAT)
    btr = beta.reshape(1, D_FEAT)

    db = _sc_degrees(nidx, hidx, attr_p).reshape(2, 2, NP, 1)
    xl1 = _tc_matmul(xp, W1)
    e1 = _tc_scale(_sc_rowpass(xl1, nidx, hidx), db)
    xl2 = _tc_fuse(_sc_rowpass(e1, hidx, nidx), db, b1r, gr, btr, W2)
    e2 = _tc_scale(_sc_rowpass(xl2, nidx, hidx), db)
    out = _tc_final(_sc_rowpass(e2, hidx, nidx), db, b2r, xp)
    return out[:N]


# speedup vs baseline: 10.8979x; 1.0346x over previous
import functools

import jax
import jax.numpy as jnp
from jax import lax
from jax.experimental import pallas as pl
from jax.experimental.pallas import tpu as pltpu
from jax.experimental.pallas import tpu_sc as plsc

N = 10000
NP = 10240
E = 320000
NW = 32
CH = 79
K = 128
EP = NW * CH * K
RT = NP // 16
D_FEAT = 128
BLK = 2560

_MESH = plsc.VectorSubcoreMesh(
    core_axis_name="c", subcore_axis_name="s", num_cores=2, num_subcores=16)


def _zero_rows(rows):
    def body(i, _):
        for j in range(8):
            rows[i, pl.ds(j * 16, 16)] = jnp.zeros((16,), jnp.float32)
        return 0
    lax.fori_loop(0, 128, body, 0)


def _rowpass_body(src, gidx, sidx, out, gvb, svb, rows, acc, gsem, isem, ssem):
    c = lax.axis_index("c")
    s = lax.axis_index("s")
    wid = s * 2 + c
    gw = gidx.at[wid]
    sw = sidx.at[wid]
    _zero_rows(rows.at[0])
    base = s * RT
    for k in range(RT // K):
        pltpu.sync_copy(rows.at[0], acc.at[pl.ds(base + k * K, K)])
    plsc.subcore_barrier()

    pltpu.sync_copy(gw.at[0], gvb.at[0])
    pltpu.sync_copy(sw.at[0], svb.at[0])
    pltpu.async_copy(src.at[gvb.at[0]], rows.at[0], gsem.at[0])
    pltpu.async_copy(gw.at[1], gvb.at[1], isem.at[1])
    pltpu.async_copy(sw.at[1], svb.at[1], isem.at[1])

    def chunk(ci, _):
        p2 = lax.rem(ci, 2)
        pn2 = 1 - p2
        p4 = lax.rem(ci, 4)

        @pl.when(ci + 1 < CH)
        def _():
            pn4 = lax.rem(ci + 1, 4)

            @pl.when(ci >= 1)
            def _():
                pltpu.make_async_copy(
                    rows.at[pn2], acc.at[svb.at[lax.rem(ci - 1, 4)]],
                    ssem.at[pn2]).wait()
            pltpu.make_async_copy(gw.at[ci + 1], gvb.at[pn4], isem.at[pn4]).wait()
            pltpu.make_async_copy(sw.at[ci + 1], svb.at[pn4], isem.at[pn4]).wait()
            pltpu.async_copy(src.at[gvb.at[pn4]], rows.at[pn2], gsem.at[pn2])

        pltpu.make_async_copy(src.at[gvb.at[p4]], rows.at[p2], gsem.at[p2]).wait()
        pltpu.async_copy(rows.at[p2], acc.at[svb.at[p4]], ssem.at[p2], add=True)

        @pl.when(ci + 2 < CH)
        def _():
            pp4 = lax.rem(ci + 2, 4)
            pltpu.async_copy(gw.at[ci + 2], gvb.at[pp4], isem.at[pp4])
            pltpu.async_copy(sw.at[ci + 2], svb.at[pp4], isem.at[pp4])
        return 0
    lax.fori_loop(0, CH, chunk, 0)
    pltpu.make_async_copy(
        rows.at[(CH - 2) % 2], acc.at[svb.at[(CH - 2) % 4]],
        ssem.at[(CH - 2) % 2]).wait()
    pltpu.make_async_copy(
        rows.at[(CH - 1) % 2], acc.at[svb.at[(CH - 1) % 4]],
        ssem.at[(CH - 1) % 2]).wait()
    plsc.subcore_barrier()
    pltpu.sync_copy(acc.at[pl.ds(base, RT)], out.at[c, pl.ds(base, RT)])


_sc_rowpass = pl.kernel(
    _rowpass_body,
    out_type=jax.ShapeDtypeStruct((2, NP, D_FEAT), jnp.float32),
    mesh=_MESH,
    scratch_types=[
        pltpu.VMEM((4, K), jnp.int32),
        pltpu.VMEM((4, K), jnp.int32),
        pltpu.VMEM((2, K, D_FEAT), jnp.float32),
        pltpu.VMEM_SHARED((NP, D_FEAT), jnp.float32),
        pltpu.SemaphoreType.DMA((2,)),
        pltpu.SemaphoreType.DMA((4,)),
        pltpu.SemaphoreType.DMA((2,)),
    ],
)


def _degrees_body(nidx, hidx, attr, out, nv, hv, vals, ones, zb, accd, accb):
    c = lax.axis_index("c")
    s = lax.axis_index("s")
    wid = s * 2 + c
    pltpu.sync_copy(nidx.at[wid], nv)
    pltpu.sync_copy(hidx.at[wid], hv)

    def zset(i, _):
        zb[pl.ds(i * 16, 16)] = jnp.zeros((16,), jnp.float32)
        return 0
    lax.fori_loop(0, 40, zset, 0)

    def oset(i, _):
        ones[pl.ds(i * 16, 16)] = jnp.full((16,), 1.0, jnp.float32)
        return 0
    lax.fori_loop(0, 8, oset, 0)

    base = s * RT
    pltpu.sync_copy(zb.at[pl.ds(0, RT)], accd.at[pl.ds(base, RT)])
    pltpu.sync_copy(zb.at[pl.ds(0, RT)], accb.at[pl.ds(base, RT)])
    plsc.subcore_barrier()

    def chunk(ci, _):
        pltpu.sync_copy(attr.at[hv.at[ci]], vals)
        pltpu.sync_copy(vals, accd.at[nv.at[ci]], add=True)
        pltpu.sync_copy(ones, accb.at[hv.at[ci]], add=True)
        return 0
    lax.fori_loop(0, CH, chunk, 0)
    plsc.subcore_barrier()
    pltpu.sync_copy(accd.at[pl.ds(base, RT)], out.at[c, 0, pl.ds(base, RT)])
    pltpu.sync_copy(accb.at[pl.ds(base, RT)], out.at[c, 1, pl.ds(base, RT)])


_sc_degrees = pl.kernel(
    _degrees_body,
    out_type=jax.ShapeDtypeStruct((2, 2, NP), jnp.float32),
    mesh=_MESH,
    scratch_types=[
        pltpu.VMEM((CH, K), jnp.int32),
        pltpu.VMEM((CH, K), jnp.int32),
        pltpu.VMEM((K,), jnp.float32),
        pltpu.VMEM((K,), jnp.float32),
        pltpu.VMEM((640,), jnp.float32),
        pltpu.VMEM_SHARED((NP,), jnp.float32),
        pltpu.VMEM_SHARED((NP,), jnp.float32),
    ],
)


def _safe_inv(d):
    return jnp.where(d == 0, 0.0, 1.0 / jnp.where(d == 0, 1.0, d))


def _mm_body(x_ref, w_ref, o_ref):
    o_ref[...] = jnp.dot(x_ref[...], w_ref[...],
                         preferred_element_type=jnp.float32)


_tc_matmul = pl.pallas_call(
    _mm_body,
    grid=(NP // BLK,),
    in_specs=[
        pl.BlockSpec((BLK, D_FEAT), lambda i: (i, 0)),
        pl.BlockSpec((D_FEAT, D_FEAT), lambda i: (0, 0)),
    ],
    out_specs=pl.BlockSpec((BLK, D_FEAT), lambda i: (i, 0)),
    out_shape=jax.ShapeDtypeStruct((NP, D_FEAT), jnp.float32),
)


def _scale_body(e_ref, db_ref, o_ref):
    ssum = e_ref[0] + e_ref[1]
    b = db_ref[0, 1] + db_ref[1, 1]
    o_ref[...] = ssum * _safe_inv(b)


_tc_scale = pl.pallas_call(
    _scale_body,
    grid=(NP // BLK,),
    in_specs=[
        pl.BlockSpec((2, BLK, D_FEAT), lambda i: (0, i, 0)),
        pl.BlockSpec((2, 2, BLK, 1), lambda i: (0, 0, i, 0)),
    ],
    out_specs=pl.BlockSpec((BLK, D_FEAT), lambda i: (i, 0)),
    out_shape=jax.ShapeDtypeStruct((NP, D_FEAT), jnp.float32),
)


def _fuse_body(p_ref, db_ref, b1_ref, g_ref, bt_ref, w_ref, o_ref):
    ssum = p_ref[0] + p_ref[1]
    d = db_ref[0, 0] + db_ref[1, 0]
    h = ssum * _safe_inv(d) + b1_ref[...]
    h = jnp.maximum(h, 0.0)
    mu = jnp.mean(h, axis=-1, keepdims=True)
    var = jnp.mean((h - mu) ** 2, axis=-1, keepdims=True)
    hn = (h - mu) / jnp.sqrt(var + 1e-5) * g_ref[...] + bt_ref[...]
    o_ref[...] = jnp.dot(hn, w_ref[...], preferred_element_type=jnp.float32)


_tc_fuse = pl.pallas_call(
    _fuse_body,
    grid=(NP // BLK,),
    in_specs=[
        pl.BlockSpec((2, BLK, D_FEAT), lambda i: (0, i, 0)),
        pl.BlockSpec((2, 2, BLK, 1), lambda i: (0, 0, i, 0)),
        pl.BlockSpec((1, D_FEAT), lambda i: (0, 0)),
        pl.BlockSpec((1, D_FEAT), lambda i: (0, 0)),
        pl.BlockSpec((1, D_FEAT), lambda i: (0, 0)),
        pl.BlockSpec((D_FEAT, D_FEAT), lambda i: (0, 0)),
    ],
    out_specs=pl.BlockSpec((BLK, D_FEAT), lambda i: (i, 0)),
    out_shape=jax.ShapeDtypeStruct((NP, D_FEAT), jnp.float32),
)


def _final_body(p_ref, db_ref, b2_ref, x_ref, o_ref):
    ssum = p_ref[0] + p_ref[1]
    d = db_ref[0, 0] + db_ref[1, 0]
    o_ref[...] = ssum * _safe_inv(d) + b2_ref[...] + x_ref[...]


_tc_final = pl.pallas_call(
    _final_body,
    grid=(NP // BLK,),
    in_specs=[
        pl.BlockSpec((2, BLK, D_FEAT), lambda i: (0, i, 0)),
        pl.BlockSpec((2, 2, BLK, 1), lambda i: (0, 0, i, 0)),
        pl.BlockSpec((1, D_FEAT), lambda i: (0, 0)),
        pl.BlockSpec((BLK, D_FEAT), lambda i: (i, 0)),
    ],
    out_specs=pl.BlockSpec((BLK, D_FEAT), lambda i: (i, 0)),
    out_shape=jax.ShapeDtypeStruct((NP, D_FEAT), jnp.float32),
)


def kernel(x, edge_index, edge_attr, W1, b1, gamma, beta, W2, b2):
    xp = jnp.pad(x, ((0, NP - N), (0, 0)))
    attr_p = jnp.pad(edge_attr, (0, NP - N))
    nidx = jnp.pad(edge_index[0], (0, EP - E), constant_values=N).reshape(NW, CH, K)
    hidx = jnp.pad(edge_index[1], (0, EP - E), constant_values=N).reshape(NW, CH, K)
    b1r = b1.reshape(1, D_FEAT)
    b2r = b2.reshape(1, D_FEAT)
    gr = gamma.reshape(1, D_FEAT)
    btr = beta.reshape(1, D_FEAT)

    db = _sc_degrees(nidx, hidx, attr_p).reshape(2, 2, NP, 1)
    xl1 = _tc_matmul(xp, W1)
    e1 = _tc_scale(_sc_rowpass(xl1, nidx, hidx), db)
    xl2 = _tc_fuse(_sc_rowpass(e1, hidx, nidx), db, b1r, gr, btr, W2)
    e2 = _tc_scale(_sc_rowpass(xl2, nidx, hidx), db)
    out = _tc_final(_sc_rowpass(e2, hidx, nidx), db, b2r, xp)
    return out[:N]


# speedup vs baseline: 20.5523x; 1.8859x over previous
import functools

import jax
import jax.numpy as jnp
from jax import lax
from jax.experimental import pallas as pl
from jax.experimental.pallas import tpu as pltpu
from jax.experimental.pallas import tpu_sc as plsc

N = 10000
NP = 10240
E = 320000
NW = 32
CH = 79
K = 128
EP = NW * CH * K
RT = NP // 16
D_FEAT = 128
BLK = 2560

_MESH = plsc.VectorSubcoreMesh(
    core_axis_name="c", subcore_axis_name="s", num_cores=2, num_subcores=16)


def _zero_rows(rows):
    def body(i, _):
        for j in range(8):
            rows[i, pl.ds(j * 16, 16)] = jnp.zeros((16,), jnp.float32)
        return 0
    lax.fori_loop(0, 128, body, 0)


def _rowpass_body(src, gidx, sidx, out, gvb, svb, rows, acc, gsem, isem, ssem):
    c = lax.axis_index("c")
    s = lax.axis_index("s")
    wid = s * 2 + c
    gw = gidx.at[wid]
    sw = sidx.at[wid]
    _zero_rows(rows.at[0])
    base = s * RT
    for k in range(RT // K):
        pltpu.sync_copy(rows.at[0], acc.at[pl.ds(base + k * K, K)])
    plsc.subcore_barrier()

    pltpu.sync_copy(gw.at[0], gvb.at[0])
    pltpu.sync_copy(sw.at[0], svb.at[0])
    pltpu.async_copy(src.at[gvb.at[0]], rows.at[0], gsem.at[0])
    pltpu.async_copy(gw.at[1], gvb.at[1], isem.at[1])
    pltpu.async_copy(sw.at[1], svb.at[1], isem.at[1])

    def chunk(ci, _):
        p2 = lax.rem(ci, 2)
        pn2 = 1 - p2
        p4 = lax.rem(ci, 4)

        @pl.when(ci + 1 < CH)
        def _():
            pn4 = lax.rem(ci + 1, 4)

            @pl.when(ci >= 1)
            def _():
                pltpu.make_async_copy(
                    rows.at[pn2], acc.at[svb.at[lax.rem(ci - 1, 4)]],
                    ssem.at[pn2]).wait()
            pltpu.make_async_copy(gw.at[ci + 1], gvb.at[pn4], isem.at[pn4]).wait()
            pltpu.make_async_copy(sw.at[ci + 1], svb.at[pn4], isem.at[pn4]).wait()
            pltpu.async_copy(src.at[gvb.at[pn4]], rows.at[pn2], gsem.at[pn2])

        pltpu.make_async_copy(src.at[gvb.at[p4]], rows.at[p2], gsem.at[p2]).wait()
        pltpu.async_copy(rows.at[p2], acc.at[svb.at[p4]], ssem.at[p2], add=True)

        @pl.when(ci + 2 < CH)
        def _():
            pp4 = lax.rem(ci + 2, 4)
            pltpu.async_copy(gw.at[ci + 2], gvb.at[pp4], isem.at[pp4])
            pltpu.async_copy(sw.at[ci + 2], svb.at[pp4], isem.at[pp4])
        return 0
    lax.fori_loop(0, CH, chunk, 0)
    pltpu.make_async_copy(
        rows.at[(CH - 2) % 2], acc.at[svb.at[(CH - 2) % 4]],
        ssem.at[(CH - 2) % 2]).wait()
    pltpu.make_async_copy(
        rows.at[(CH - 1) % 2], acc.at[svb.at[(CH - 1) % 4]],
        ssem.at[(CH - 1) % 2]).wait()
    plsc.subcore_barrier()
    pltpu.sync_copy(acc.at[pl.ds(base, RT)], out.at[c, pl.ds(base, RT)])


_sc_rowpass = pl.kernel(
    _rowpass_body,
    out_type=jax.ShapeDtypeStruct((2, NP, D_FEAT), jnp.float32),
    mesh=_MESH,
    scratch_types=[
        pltpu.VMEM((4, K), jnp.int32),
        pltpu.VMEM((4, K), jnp.int32),
        pltpu.VMEM((2, K, D_FEAT), jnp.float32),
        pltpu.VMEM_SHARED((NP, D_FEAT), jnp.float32),
        pltpu.SemaphoreType.DMA((2,)),
        pltpu.SemaphoreType.DMA((4,)),
        pltpu.SemaphoreType.DMA((2,)),
    ],
)


def _degrees_body(nidx, hidx, attr, out, nv, hv, vals, ones, zb, accd, accb):
    c = lax.axis_index("c")
    s = lax.axis_index("s")
    wid = s * 2 + c
    pltpu.sync_copy(nidx.at[wid], nv)
    pltpu.sync_copy(hidx.at[wid], hv)

    def zset(i, _):
        zb[pl.ds(i * 16, 16)] = jnp.zeros((16,), jnp.float32)
        return 0
    lax.fori_loop(0, 40, zset, 0)

    def oset(i, _):
        ones[pl.ds(i * 16, 16)] = jnp.full((16,), 1.0, jnp.float32)
        return 0
    lax.fori_loop(0, 8, oset, 0)

    base = s * RT
    pltpu.sync_copy(zb.at[pl.ds(0, RT)], accd.at[pl.ds(base, RT)])
    pltpu.sync_copy(zb.at[pl.ds(0, RT)], accb.at[pl.ds(base, RT)])
    plsc.subcore_barrier()

    def chunk(ci, _):
        pltpu.sync_copy(attr.at[hv.at[ci]], vals)
        pltpu.sync_copy(vals, accd.at[nv.at[ci]], add=True)
        pltpu.sync_copy(ones, accb.at[hv.at[ci]], add=True)
        return 0
    lax.fori_loop(0, CH, chunk, 0)
    plsc.subcore_barrier()
    pltpu.sync_copy(accd.at[pl.ds(base, RT)], out.at[c, 0, pl.ds(base, RT)])
    pltpu.sync_copy(accb.at[pl.ds(base, RT)], out.at[c, 1, pl.ds(base, RT)])


_sc_degrees = pl.kernel(
    _degrees_body,
    out_type=jax.ShapeDtypeStruct((2, 2, NP), jnp.float32),
    mesh=_MESH,
    scratch_types=[
        pltpu.VMEM((CH, K), jnp.int32),
        pltpu.VMEM((CH, K), jnp.int32),
        pltpu.VMEM((K,), jnp.float32),
        pltpu.VMEM((K,), jnp.float32),
        pltpu.VMEM((640,), jnp.float32),
        pltpu.VMEM_SHARED((NP,), jnp.float32),
        pltpu.VMEM_SHARED((NP,), jnp.float32),
    ],
)


def _safe_inv(d):
    return jnp.where(d == 0, 0.0, 1.0 / jnp.where(d == 0, 1.0, d))


def _mm_body(x_ref, w_ref, o_ref):
    o_ref[...] = jnp.dot(x_ref[...], w_ref[...],
                         preferred_element_type=jnp.float32)


_tc_matmul = pl.pallas_call(
    _mm_body,
    grid=(NP // BLK,),
    in_specs=[
        pl.BlockSpec((BLK, D_FEAT), lambda i: (i, 0)),
        pl.BlockSpec((D_FEAT, D_FEAT), lambda i: (0, 0)),
    ],
    out_specs=pl.BlockSpec((BLK, D_FEAT), lambda i: (i, 0)),
    out_shape=jax.ShapeDtypeStruct((NP, D_FEAT), jnp.float32),
)


def _scale_body(e_ref, db_ref, o_ref):
    ssum = e_ref[0] + e_ref[1]
    b = db_ref[0, 1] + db_ref[1, 1]
    o_ref[...] = ssum * _safe_inv(b)


_tc_scale = pl.pallas_call(
    _scale_body,
    grid=(NP // BLK,),
    in_specs=[
        pl.BlockSpec((2, BLK, D_FEAT), lambda i: (0, i, 0)),
        pl.BlockSpec((2, 2, BLK, 1), lambda i: (0, 0, i, 0)),
    ],
    out_specs=pl.BlockSpec((BLK, D_FEAT), lambda i: (i, 0)),
    out_shape=jax.ShapeDtypeStruct((NP, D_FEAT), jnp.float32),
)


def _fuse_body(p_ref, db_ref, b1_ref, g_ref, bt_ref, w_ref, o_ref):
    ssum = p_ref[0] + p_ref[1]
    d = db_ref[0, 0] + db_ref[1, 0]
    h = ssum * _safe_inv(d) + b1_ref[...]
    h = jnp.maximum(h, 0.0)
    mu = jnp.mean(h, axis=-1, keepdims=True)
    var = jnp.mean((h - mu) ** 2, axis=-1, keepdims=True)
    hn = (h - mu) / jnp.sqrt(var + 1e-5) * g_ref[...] + bt_ref[...]
    o_ref[...] = jnp.dot(hn, w_ref[...], preferred_element_type=jnp.float32)


_tc_fuse = pl.pallas_call(
    _fuse_body,
    grid=(NP // BLK,),
    in_specs=[
        pl.BlockSpec((2, BLK, D_FEAT), lambda i: (0, i, 0)),
        pl.BlockSpec((2, 2, BLK, 1), lambda i: (0, 0, i, 0)),
        pl.BlockSpec((1, D_FEAT), lambda i: (0, 0)),
        pl.BlockSpec((1, D_FEAT), lambda i: (0, 0)),
        pl.BlockSpec((1, D_FEAT), lambda i: (0, 0)),
        pl.BlockSpec((D_FEAT, D_FEAT), lambda i: (0, 0)),
    ],
    out_specs=pl.BlockSpec((BLK, D_FEAT), lambda i: (i, 0)),
    out_shape=jax.ShapeDtypeStruct((NP, D_FEAT), jnp.float32),
)


def _final_body(p_ref, db_ref, b2_ref, x_ref, o_ref):
    ssum = p_ref[0] + p_ref[1]
    d = db_ref[0, 0] + db_ref[1, 0]
    o_ref[...] = ssum * _safe_inv(d) + b2_ref[...] + x_ref[...]


_tc_final = pl.pallas_call(
    _final_body,
    grid=(NP // BLK,),
    in_specs=[
        pl.BlockSpec((2, BLK, D_FEAT), lambda i: (0, i, 0)),
        pl.BlockSpec((2, 2, BLK, 1), lambda i: (0, 0, i, 0)),
        pl.BlockSpec((1, D_FEAT), lambda i: (0, 0)),
        pl.BlockSpec((BLK, D_FEAT), lambda i: (i, 0)),
    ],
    out_specs=pl.BlockSpec((BLK, D_FEAT), lambda i: (i, 0)),
    out_shape=jax.ShapeDtypeStruct((NP, D_FEAT), jnp.float32),
)


def kernel(x, edge_index, edge_attr, W1, b1, gamma, beta, W2, b2):
    xp = jnp.pad(x, ((0, NP - N), (0, 0)))
    attr_p = jnp.pad(edge_attr, (0, NP - N))
    pad_ix = N + jnp.arange(EP - E, dtype=jnp.int32) % (NP - N)
    nidx = jnp.concatenate([edge_index[0], pad_ix]).reshape(NW, CH, K)
    hidx = jnp.concatenate([edge_index[1], pad_ix]).reshape(NW, CH, K)
    b1r = b1.reshape(1, D_FEAT)
    b2r = b2.reshape(1, D_FEAT)
    gr = gamma.reshape(1, D_FEAT)
    btr = beta.reshape(1, D_FEAT)

    db = _sc_degrees(nidx, hidx, attr_p).reshape(2, 2, NP, 1)
    xl1 = _tc_matmul(xp, W1)
    e1 = _tc_scale(_sc_rowpass(xl1, nidx, hidx), db)
    xl2 = _tc_fuse(_sc_rowpass(e1, hidx, nidx), db, b1r, gr, btr, W2)
    e2 = _tc_scale(_sc_rowpass(xl2, nidx, hidx), db)
    out = _tc_final(_sc_rowpass(e2, hidx, nidx), db, b2r, xp)
    return out[:N]


# speedup vs baseline: 22.8873x; 1.1136x over previous
import functools

import jax
import jax.numpy as jnp
from jax import lax
from jax.experimental import pallas as pl
from jax.experimental.pallas import tpu as pltpu
from jax.experimental.pallas import tpu_sc as plsc

N = 10000
NP = 10240
E = 320000
NW = 32
CH = 79
K = 128
EP = NW * CH * K
RT = NP // 16
D_FEAT = 128
BLK = 2560

_MESH = plsc.VectorSubcoreMesh(
    core_axis_name="c", subcore_axis_name="s", num_cores=2, num_subcores=16)


def _zero_rows(rows):
    def body(i, _):
        for j in range(8):
            rows[i, pl.ds(j * 16, 16)] = jnp.zeros((16,), jnp.float32)
        return 0
    lax.fori_loop(0, 128, body, 0)


def _rowpass_body(src, gidx, sidx, out, gvb, svb, rows, acc, gsem, isem, ssem):
    c = lax.axis_index("c")
    s = lax.axis_index("s")
    wid = s * 2 + c
    gw = gidx.at[wid]
    sw = sidx.at[wid]
    _zero_rows(rows.at[0])
    base = s * RT
    for k in range(RT // K):
        pltpu.sync_copy(rows.at[0], acc.at[pl.ds(base + k * K, K)])
    plsc.subcore_barrier()

    pltpu.sync_copy(gw.at[0], gvb.at[0])
    pltpu.sync_copy(sw.at[0], svb.at[0])
    pltpu.async_copy(src.at[gvb.at[0]], rows.at[0], gsem.at[0])
    pltpu.async_copy(gw.at[1], gvb.at[1], isem.at[1])
    pltpu.async_copy(sw.at[1], svb.at[1], isem.at[1])

    def chunk(ci, _):
        p2 = lax.rem(ci, 2)
        pn2 = 1 - p2
        p4 = lax.rem(ci, 4)

        @pl.when(ci + 1 < CH)
        def _():
            pn4 = lax.rem(ci + 1, 4)

            @pl.when(ci >= 1)
            def _():
                pltpu.make_async_copy(
                    rows.at[pn2], acc.at[svb.at[lax.rem(ci - 1, 4)]],
                    ssem.at[pn2]).wait()
            pltpu.make_async_copy(gw.at[ci + 1], gvb.at[pn4], isem.at[pn4]).wait()
            pltpu.make_async_copy(sw.at[ci + 1], svb.at[pn4], isem.at[pn4]).wait()
            pltpu.async_copy(src.at[gvb.at[pn4]], rows.at[pn2], gsem.at[pn2])

        pltpu.make_async_copy(src.at[gvb.at[p4]], rows.at[p2], gsem.at[p2]).wait()
        pltpu.async_copy(rows.at[p2], acc.at[svb.at[p4]], ssem.at[p2], add=True)

        @pl.when(ci + 2 < CH)
        def _():
            pp4 = lax.rem(ci + 2, 4)
            pltpu.async_copy(gw.at[ci + 2], gvb.at[pp4], isem.at[pp4])
            pltpu.async_copy(sw.at[ci + 2], svb.at[pp4], isem.at[pp4])
        return 0
    lax.fori_loop(0, CH, chunk, 0)
    pltpu.make_async_copy(
        rows.at[(CH - 2) % 2], acc.at[svb.at[(CH - 2) % 4]],
        ssem.at[(CH - 2) % 2]).wait()
    pltpu.make_async_copy(
        rows.at[(CH - 1) % 2], acc.at[svb.at[(CH - 1) % 4]],
        ssem.at[(CH - 1) % 2]).wait()
    plsc.subcore_barrier()
    pltpu.sync_copy(acc.at[pl.ds(base, RT)], out.at[c, pl.ds(base, RT)])


_sc_rowpass = pl.kernel(
    _rowpass_body,
    out_type=jax.ShapeDtypeStruct((2, NP, D_FEAT), jnp.float32),
    mesh=_MESH,
    scratch_types=[
        pltpu.VMEM((4, K), jnp.int32),
        pltpu.VMEM((4, K), jnp.int32),
        pltpu.VMEM((2, K, D_FEAT), jnp.float32),
        pltpu.VMEM_SHARED((NP, D_FEAT), jnp.float32),
        pltpu.SemaphoreType.DMA((2,)),
        pltpu.SemaphoreType.DMA((4,)),
        pltpu.SemaphoreType.DMA((2,)),
    ],
)


def _rowpass_deg_body(src, gidx, sidx, attr, out, out_db, gvb, svb, rows, acc,
                      gsem, isem, ssem, aval, ones, zb, accd, accb,
                      avsem, dsem, bsem):
    c = lax.axis_index("c")
    s = lax.axis_index("s")
    wid = s * 2 + c
    gw = gidx.at[wid]
    sw = sidx.at[wid]
    _zero_rows(rows.at[0])
    base = s * RT
    for k in range(RT // K):
        pltpu.sync_copy(rows.at[0], acc.at[pl.ds(base + k * K, K)])

    def zset(i, _):
        zb[pl.ds(i * 16, 16)] = jnp.zeros((16,), jnp.float32)
        return 0
    lax.fori_loop(0, 40, zset, 0)

    def oset(i, _):
        ones[pl.ds(i * 16, 16)] = jnp.full((16,), 1.0, jnp.float32)
        return 0
    lax.fori_loop(0, 8, oset, 0)
    pltpu.sync_copy(zb.at[pl.ds(0, RT)], accd.at[pl.ds(base, RT)])
    pltpu.sync_copy(zb.at[pl.ds(0, RT)], accb.at[pl.ds(base, RT)])
    plsc.subcore_barrier()

    pltpu.sync_copy(gw.at[0], gvb.at[0])
    pltpu.sync_copy(sw.at[0], svb.at[0])
    pltpu.async_copy(src.at[gvb.at[0]], rows.at[0], gsem.at[0])
    pltpu.async_copy(attr.at[svb.at[0]], aval.at[0], avsem.at[0])
    pltpu.async_copy(gw.at[1], gvb.at[1], isem.at[1])
    pltpu.async_copy(sw.at[1], svb.at[1], isem.at[1])

    def chunk(ci, _):
        p2 = lax.rem(ci, 2)
        pn2 = 1 - p2
        p4 = lax.rem(ci, 4)

        @pl.when(ci + 1 < CH)
        def _():
            pn4 = lax.rem(ci + 1, 4)

            @pl.when(ci >= 1)
            def _():
                pltpu.make_async_copy(
                    rows.at[pn2], acc.at[svb.at[lax.rem(ci - 1, 4)]],
                    ssem.at[pn2]).wait()
            pltpu.make_async_copy(gw.at[ci + 1], gvb.at[pn4], isem.at[pn4]).wait()
            pltpu.make_async_copy(sw.at[ci + 1], svb.at[pn4], isem.at[pn4]).wait()
            pltpu.async_copy(src.at[gvb.at[pn4]], rows.at[pn2], gsem.at[pn2])
            pltpu.async_copy(attr.at[svb.at[pn4]], aval.at[pn4], avsem.at[pn4])

        @pl.when(ci >= 1)
        def _():
            j4 = lax.rem(ci - 1, 4)
            pltpu.make_async_copy(attr.at[svb.at[j4]], aval.at[j4],
                                  avsem.at[j4]).wait()
            pltpu.async_copy(aval.at[j4], accd.at[gvb.at[j4]], dsem.at[j4],
                             add=True)
            pltpu.async_copy(ones, accb.at[svb.at[j4]], bsem.at[j4], add=True)

        pltpu.make_async_copy(src.at[gvb.at[p4]], rows.at[p2], gsem.at[p2]).wait()
        pltpu.async_copy(rows.at[p2], acc.at[svb.at[p4]], ssem.at[p2], add=True)

        @pl.when(ci + 2 < CH)
        def _():
            pp4 = lax.rem(ci + 2, 4)

            @pl.when(ci >= 2)
            def _():
                q4 = lax.rem(ci - 2, 4)
                pltpu.make_async_copy(aval.at[q4], accd.at[gvb.at[q4]],
                                      dsem.at[q4]).wait()
                pltpu.make_async_copy(ones, accb.at[svb.at[q4]],
                                      bsem.at[q4]).wait()
            pltpu.async_copy(gw.at[ci + 2], gvb.at[pp4], isem.at[pp4])
            pltpu.async_copy(sw.at[ci + 2], svb.at[pp4], isem.at[pp4])
        return 0
    lax.fori_loop(0, CH, chunk, 0)
    pltpu.make_async_copy(
        rows.at[(CH - 2) % 2], acc.at[svb.at[(CH - 2) % 4]],
        ssem.at[(CH - 2) % 2]).wait()
    pltpu.make_async_copy(
        rows.at[(CH - 1) % 2], acc.at[svb.at[(CH - 1) % 4]],
        ssem.at[(CH - 1) % 2]).wait()
    for j in (CH - 3, CH - 2):
        pltpu.make_async_copy(aval.at[j % 4], accd.at[gvb.at[j % 4]],
                              dsem.at[j % 4]).wait()
        pltpu.make_async_copy(ones, accb.at[svb.at[j % 4]],
                              bsem.at[j % 4]).wait()
    jl = (CH - 1) % 4
    pltpu.make_async_copy(attr.at[svb.at[jl]], aval.at[jl], avsem.at[jl]).wait()
    pltpu.async_copy(aval.at[jl], accd.at[gvb.at[jl]], dsem.at[jl], add=True)
    pltpu.async_copy(ones, accb.at[svb.at[jl]], bsem.at[jl], add=True)
    pltpu.make_async_copy(aval.at[jl], accd.at[gvb.at[jl]], dsem.at[jl]).wait()
    pltpu.make_async_copy(ones, accb.at[svb.at[jl]], bsem.at[jl]).wait()
    plsc.subcore_barrier()
    pltpu.sync_copy(acc.at[pl.ds(base, RT)], out.at[c, pl.ds(base, RT)])
    pltpu.sync_copy(accd.at[pl.ds(base, RT)], out_db.at[c, 0, pl.ds(base, RT)])
    pltpu.sync_copy(accb.at[pl.ds(base, RT)], out_db.at[c, 1, pl.ds(base, RT)])


_sc_rowpass_deg = pl.kernel(
    _rowpass_deg_body,
    out_type=[
        jax.ShapeDtypeStruct((2, NP, D_FEAT), jnp.float32),
        jax.ShapeDtypeStruct((2, 2, NP), jnp.float32),
    ],
    mesh=_MESH,
    scratch_types=[
        pltpu.VMEM((4, K), jnp.int32),
        pltpu.VMEM((4, K), jnp.int32),
        pltpu.VMEM((2, K, D_FEAT), jnp.float32),
        pltpu.VMEM_SHARED((NP, D_FEAT), jnp.float32),
        pltpu.SemaphoreType.DMA((2,)),
        pltpu.SemaphoreType.DMA((4,)),
        pltpu.SemaphoreType.DMA((2,)),
        pltpu.VMEM((4, K), jnp.float32),
        pltpu.VMEM((K,), jnp.float32),
        pltpu.VMEM((RT,), jnp.float32),
        pltpu.VMEM_SHARED((NP,), jnp.float32),
        pltpu.VMEM_SHARED((NP,), jnp.float32),
        pltpu.SemaphoreType.DMA((4,)),
        pltpu.SemaphoreType.DMA((4,)),
        pltpu.SemaphoreType.DMA((4,)),
    ],
)


def _safe_inv(d):
    return jnp.where(d == 0, 0.0, 1.0 / jnp.where(d == 0, 1.0, d))


def _mm_body(x_ref, w_ref, o_ref):
    o_ref[...] = jnp.dot(x_ref[...], w_ref[...],
                         preferred_element_type=jnp.float32)


_tc_matmul = pl.pallas_call(
    _mm_body,
    grid=(NP // BLK,),
    in_specs=[
        pl.BlockSpec((BLK, D_FEAT), lambda i: (i, 0)),
        pl.BlockSpec((D_FEAT, D_FEAT), lambda i: (0, 0)),
    ],
    out_specs=pl.BlockSpec((BLK, D_FEAT), lambda i: (i, 0)),
    out_shape=jax.ShapeDtypeStruct((NP, D_FEAT), jnp.float32),
)


def _scale_body(e_ref, db_ref, o_ref):
    ssum = e_ref[0] + e_ref[1]
    b = db_ref[0, 1] + db_ref[1, 1]
    o_ref[...] = ssum * _safe_inv(b)


_tc_scale = pl.pallas_call(
    _scale_body,
    grid=(NP // BLK,),
    in_specs=[
        pl.BlockSpec((2, BLK, D_FEAT), lambda i: (0, i, 0)),
        pl.BlockSpec((2, 2, BLK, 1), lambda i: (0, 0, i, 0)),
    ],
    out_specs=pl.BlockSpec((BLK, D_FEAT), lambda i: (i, 0)),
    out_shape=jax.ShapeDtypeStruct((NP, D_FEAT), jnp.float32),
)


def _fuse_body(p_ref, db_ref, b1_ref, g_ref, bt_ref, w_ref, o_ref):
    ssum = p_ref[0] + p_ref[1]
    d = db_ref[0, 0] + db_ref[1, 0]
    h = ssum * _safe_inv(d) + b1_ref[...]
    h = jnp.maximum(h, 0.0)
    mu = jnp.mean(h, axis=-1, keepdims=True)
    var = jnp.mean((h - mu) ** 2, axis=-1, keepdims=True)
    hn = (h - mu) / jnp.sqrt(var + 1e-5) * g_ref[...] + bt_ref[...]
    o_ref[...] = jnp.dot(hn, w_ref[...], preferred_element_type=jnp.float32)


_tc_fuse = pl.pallas_call(
    _fuse_body,
    grid=(NP // BLK,),
    in_specs=[
        pl.BlockSpec((2, BLK, D_FEAT), lambda i: (0, i, 0)),
        pl.BlockSpec((2, 2, BLK, 1), lambda i: (0, 0, i, 0)),
        pl.BlockSpec((1, D_FEAT), lambda i: (0, 0)),
        pl.BlockSpec((1, D_FEAT), lambda i: (0, 0)),
        pl.BlockSpec((1, D_FEAT), lambda i: (0, 0)),
        pl.BlockSpec((D_FEAT, D_FEAT), lambda i: (0, 0)),
    ],
    out_specs=pl.BlockSpec((BLK, D_FEAT), lambda i: (i, 0)),
    out_shape=jax.ShapeDtypeStruct((NP, D_FEAT), jnp.float32),
)


def _final_body(p_ref, db_ref, b2_ref, x_ref, o_ref):
    ssum = p_ref[0] + p_ref[1]
    d = db_ref[0, 0] + db_ref[1, 0]
    o_ref[...] = ssum * _safe_inv(d) + b2_ref[...] + x_ref[...]


_tc_final = pl.pallas_call(
    _final_body,
    grid=(NP // BLK,),
    in_specs=[
        pl.BlockSpec((2, BLK, D_FEAT), lambda i: (0, i, 0)),
        pl.BlockSpec((2, 2, BLK, 1), lambda i: (0, 0, i, 0)),
        pl.BlockSpec((1, D_FEAT), lambda i: (0, 0)),
        pl.BlockSpec((BLK, D_FEAT), lambda i: (i, 0)),
    ],
    out_specs=pl.BlockSpec((BLK, D_FEAT), lambda i: (i, 0)),
    out_shape=jax.ShapeDtypeStruct((NP, D_FEAT), jnp.float32),
)


def kernel(x, edge_index, edge_attr, W1, b1, gamma, beta, W2, b2):
    xp = jnp.pad(x, ((0, NP - N), (0, 0)))
    attr_p = jnp.pad(edge_attr, (0, NP - N))
    pad_ix = N + jnp.arange(EP - E, dtype=jnp.int32) % (NP - N)
    nidx = jnp.concatenate([edge_index[0], pad_ix]).reshape(NW, CH, K)
    hidx = jnp.concatenate([edge_index[1], pad_ix]).reshape(NW, CH, K)
    b1r = b1.reshape(1, D_FEAT)
    b2r = b2.reshape(1, D_FEAT)
    gr = gamma.reshape(1, D_FEAT)
    btr = beta.reshape(1, D_FEAT)

    xl1 = _tc_matmul(xp, W1)
    part1, db = _sc_rowpass_deg(xl1, nidx, hidx, attr_p)
    db = db.reshape(2, 2, NP, 1)
    e1 = _tc_scale(part1, db)
    xl2 = _tc_fuse(_sc_rowpass(e1, hidx, nidx), db, b1r, gr, btr, W2)
    e2 = _tc_scale(_sc_rowpass(xl2, nidx, hidx), db)
    out = _tc_final(_sc_rowpass(e2, hidx, nidx), db, b2r, xp)
    return out[:N]
